# baseline (speedup 1.0000x reference)
import functools

import jax
import jax.numpy as jnp
from jax import lax
from jax.experimental import pallas as pl
from jax.experimental.pallas import tpu as pltpu
from jax.experimental.pallas import tpu_sc as plsc

N = 10000
E = 320000
D_IN = 165

N_PAD = 10240
TRASH = N
NTILES = 32
CHUNK = 128
RC = E // CHUNK
BASE = RC // NTILES
EXTRA = RC - BASE * NTILES
CH = 80
ROWS_PER_TILE = N_PAD // 16
RB = 400
_SC_PARAMS = pltpu.CompilerParams(use_tc_tiling_on_sc=False)


def _preload_packed(packed_hbm, pads_hbm, idx_p, t):
    pltpu.sync_copy(packed_hbm.at[0, pl.ds(t * BASE * CHUNK, BASE * CHUNK)],
                    idx_p.at[pl.ds(0, BASE * CHUNK)])

    @pl.when(t < EXTRA)
    def _():
        pltpu.sync_copy(
            packed_hbm.at[0, pl.ds((BASE * NTILES + t) * CHUNK, CHUNK)],
            idx_p.at[pl.ds(BASE * CHUNK, CHUNK)])

    @pl.when(t >= EXTRA)
    def _():
        pltpu.sync_copy(pads_hbm, idx_p.at[pl.ds(BASE * CHUNK, CHUNK)])

    for j in range(BASE + 1, CH):
        pltpu.sync_copy(pads_hbm, idx_p.at[pl.ds(j * CHUNK, CHUNK)])


def _degree_kernel():
    mesh = plsc.VectorSubcoreMesh(core_axis_name="c", subcore_axis_name="s")

    @functools.partial(
        pl.kernel,
        out_type=jax.ShapeDtypeStruct((2, N_PAD, 16), jnp.float32),
        mesh=mesh,
        compiler_params=_SC_PARAMS,
        scratch_types=[
            pltpu.VMEM((CH * CHUNK,), jnp.int32),
            pltpu.VMEM((4, CHUNK), jnp.int32),
            pltpu.VMEM((CHUNK, 16), jnp.float32),
            pltpu.VMEM_SHARED((N_PAD, 16), jnp.float32),
            pltpu.SemaphoreType.DMA,
            pltpu.SemaphoreType.DMA,
            pltpu.SemaphoreType.DMA,
            pltpu.SemaphoreType.DMA,
        ],
    )
    def deg(packed_hbm, ones_hbm, zeros_hbm, pads_hbm, out_hbm,
            idx_p, st_c, buf, acc, s0, s1, s2, s3):
        sems = [s0, s1, s2, s3]
        cid = lax.axis_index("c")
        sid = lax.axis_index("s")
        t = cid * 16 + sid

        pltpu.sync_copy(zeros_hbm, buf)
        for j in range(ROWS_PER_TILE // CHUNK):
            pltpu.sync_copy(buf, acc.at[pl.ds(sid * ROWS_PER_TILE + j * CHUNK, CHUNK)])
        _preload_packed(packed_hbm, pads_hbm, idx_p, t)
        pltpu.sync_copy(ones_hbm, buf)
        plsc.subcore_barrier()

        def unpack_c(i, k):
            for j in range(CHUNK // 16):
                v = idx_p[pl.ds(i * CHUNK + j * 16, 16)]
                st_c[k, pl.ds(j * 16, 16)] = jnp.right_shift(v, 16)

        def issue(k):
            pltpu.async_copy(buf, acc.at[st_c.at[k]], sems[k], add=True)

        def drain(k):
            pltpu.make_async_copy(buf, acc.at[st_c.at[k]], sems[k]).wait()

        for k in range(4):
            unpack_c(k, k)
            issue(k)

        def body(s, carry):
            for k in range(4):
                drain(k)
                unpack_c(4 * s + k, k)
                issue(k)
            return carry

        lax.fori_loop(1, CH // 4, body, 0)
        for k in range(4):
            drain(k)
        plsc.subcore_barrier()

        for j in range(ROWS_PER_TILE // CHUNK):
            off = sid * ROWS_PER_TILE + j * CHUNK
            pltpu.sync_copy(acc.at[pl.ds(off, CHUNK)], buf)
            pltpu.sync_copy(buf, out_hbm.at[cid, pl.ds(off, CHUNK)])

    return deg


def _mp_kernel(D):
    mesh = plsc.VectorSubcoreMesh(core_axis_name="c", subcore_axis_name="s")

    @functools.partial(
        pl.kernel,
        out_type=jax.ShapeDtypeStruct((2, N_PAD, D), jnp.float32),
        mesh=mesh,
        compiler_params=_SC_PARAMS,
        scratch_types=[
            pltpu.VMEM((CH * CHUNK,), jnp.int32),
            pltpu.VMEM((2, CHUNK), jnp.int32),
            pltpu.VMEM((2, CHUNK), jnp.int32),
            pltpu.VMEM((CHUNK, D), jnp.float32),
            pltpu.VMEM((CHUNK, D), jnp.float32),
            pltpu.VMEM_SHARED((N_PAD, D), jnp.float32),
            pltpu.SemaphoreType.DMA,
            pltpu.SemaphoreType.DMA,
            pltpu.SemaphoreType.DMA,
            pltpu.SemaphoreType.DMA,
        ],
    )
    def mp(packed_hbm, table_hbm, zeros_hbm, pads_hbm, out_hbm,
           idx_p, st_r, st_c, r0, r1, acc, g0, g1, t0, t1):
        rows = [r0, r1]
        semg = [g0, g1]
        sems = [t0, t1]
        cid = lax.axis_index("c")
        sid = lax.axis_index("s")
        t = cid * 16 + sid

        _preload_packed(packed_hbm, pads_hbm, idx_p, t)

        pltpu.sync_copy(zeros_hbm, r0)
        for j in range(ROWS_PER_TILE // CHUNK):
            pltpu.sync_copy(r0, acc.at[pl.ds(sid * ROWS_PER_TILE + j * CHUNK, CHUNK)])
        plsc.subcore_barrier()

        def unpack(i, b):
            for j in range(CHUNK // 16):
                v = idx_p[pl.ds(i * CHUNK + j * 16, 16)]
                st_r[b, pl.ds(j * 16, 16)] = jnp.bitwise_and(v, 0xFFFF)
                st_c[b, pl.ds(j * 16, 16)] = jnp.right_shift(v, 16)

        def issue_gather(b):
            pltpu.async_copy(table_hbm.at[st_r.at[b]], rows[b], semg[b])

        def wait_gather(b):
            pltpu.make_async_copy(table_hbm.at[st_r.at[b]], rows[b],
                                  semg[b]).wait()

        def issue_scatter(b):
            pltpu.async_copy(rows[b], acc.at[st_c.at[b]], sems[b], add=True)

        def wait_scatter(b):
            pltpu.make_async_copy(rows[b], acc.at[st_c.at[b]], sems[b]).wait()

        unpack(0, 0)
        issue_gather(0)
        unpack(1, 1)
        issue_gather(1)

        def body(s, carry):
            for k in range(2):
                i = 2 * s + k
                wait_gather(k)
                issue_scatter(k)
                wait_scatter(k)
                unpack(i + 2, k)
                issue_gather(k)
            return carry

        lax.fori_loop(0, (CH - 2) // 2, body, 0)

        for k in range(2):
            wait_gather(k)
            issue_scatter(k)
        for k in range(2):
            wait_scatter(k)
        plsc.subcore_barrier()

        for j in range(ROWS_PER_TILE // CHUNK):
            off = sid * ROWS_PER_TILE + j * CHUNK
            pltpu.sync_copy(acc.at[pl.ds(off, CHUNK)], r0)
            pltpu.sync_copy(r0, out_hbm.at[cid, pl.ds(off, CHUNK)])

    return mp


def _dis(degp0, degp1):
    deg = degp0[:, 0:1] + degp1[:, 0:1] + 1.0
    return lax.rsqrt(deg)


def _tc_pack(e_ref, out_ref):
    out_ref[...] = e_ref[0:1, :] + e_ref[1:2, :] * 65536


def _tc_a(degp_ref, x_ref, w_ref, out_ref):
    dis = _dis(degp_ref[0], degp_ref[1])
    out_ref[...] = dis * jnp.dot(x_ref[...], w_ref[...],
                                 preferred_element_type=jnp.float32)


def _tc_b(degp_ref, acc_ref, tab_ref, b_ref, w_ref, out_ref):
    dis = _dis(degp_ref[0], degp_ref[1])
    s = acc_ref[0] + acc_ref[1] + tab_ref[...]
    h = jnp.maximum(dis * s + b_ref[...], 0.0)
    out_ref[...] = dis * jnp.dot(h, w_ref[...],
                                 preferred_element_type=jnp.float32)


def _tc_c(degp_ref, acc_ref, tab_ref, b_ref, w_ref, bd_ref, out_ref):
    dis = _dis(degp_ref[0], degp_ref[1])
    s = acc_ref[0] + acc_ref[1] + tab_ref[...]
    emb = jnp.maximum(dis * s + b_ref[...], 0.0)
    out_ref[...] = jnp.dot(emb, w_ref[...],
                           preferred_element_type=jnp.float32) + bd_ref[...]


def _row_blocked(d):
    return pl.BlockSpec((RB, d), lambda i: (i, 0))


def _deg_spec():
    return pl.BlockSpec((2, RB, 16), lambda i: (0, i, 0))


def _acc_spec(d):
    return pl.BlockSpec((2, RB, d), lambda i: (0, i, 0))


def _full(shape):
    return pl.BlockSpec(shape, lambda i: tuple(0 for _ in shape))


@jax.jit
def kernel(x, edge_index, W1, b1, W2, b2, Wd, bd):
    f32 = jnp.float32
    i32 = jnp.int32
    pads_p = jnp.full((CHUNK,), TRASH * 65536, i32)

    ones16 = jnp.ones((CHUNK, 16), f32)
    zeros16 = jnp.zeros((CHUNK, 16), f32)
    zeros128 = jnp.zeros((CHUNK, 128), f32)
    zeros64 = jnp.zeros((CHUNK, 64), f32)

    eblk = E // 25
    packed = pl.pallas_call(
        _tc_pack,
        grid=(25,),
        in_specs=[pl.BlockSpec((2, eblk), lambda i: (0, i))],
        out_specs=pl.BlockSpec((1, eblk), lambda i: (0, i)),
        out_shape=jax.ShapeDtypeStruct((1, E), i32),
    )(edge_index)

    degp = _degree_kernel()(packed, ones16, zeros16, pads_p)

    grid = (N // RB,)
    table1 = pl.pallas_call(
        _tc_a,
        grid=grid,
        in_specs=[_deg_spec(), _row_blocked(D_IN), _full((D_IN, 128))],
        out_specs=_row_blocked(128),
        out_shape=jax.ShapeDtypeStruct((N, 128), f32),
    )(degp, x, W1.T)

    acc1 = _mp_kernel(128)(packed, table1, zeros128, pads_p)

    table2 = pl.pallas_call(
        _tc_b,
        grid=grid,
        in_specs=[_deg_spec(), _acc_spec(128), _row_blocked(128),
                  _full((1, 128)), _full((128, 64))],
        out_specs=_row_blocked(64),
        out_shape=jax.ShapeDtypeStruct((N, 64), f32),
    )(degp, acc1, table1, b1.reshape(1, 128), W2.T)

    acc2 = _mp_kernel(64)(packed, table2, zeros64, pads_p)

    dout = 256
    wdt = jnp.zeros((64, dout), f32).at[:, :D_IN].set(Wd.T)
    bd_pad = jnp.zeros((1, dout), f32).at[0, :D_IN].set(bd)
    out = pl.pallas_call(
        _tc_c,
        grid=grid,
        in_specs=[_deg_spec(), _acc_spec(64), _row_blocked(64),
                  _full((1, 64)), _full((64, dout)), _full((1, dout))],
        out_specs=_row_blocked(dout),
        out_shape=jax.ShapeDtypeStruct((N, dout), f32),
    )(degp, acc2, table2, b2.reshape(1, 64), wdt, bd_pad)

    return out[:, :D_IN]


# speedup vs baseline: 1.1799x; 1.1799x over previous
import functools

import jax
import jax.numpy as jnp
from jax import lax
from jax.experimental import pallas as pl
from jax.experimental.pallas import tpu as pltpu
from jax.experimental.pallas import tpu_sc as plsc

N = 10000
E = 320000
D_IN = 165

N_PAD = 10240
TRASH = N
NTILES = 32
CHUNK = 128
RC = E // CHUNK
BASE = RC // NTILES
EXTRA = RC - BASE * NTILES
CH = 80
ROWS_PER_TILE = N_PAD // 16
RB = 400
_SC_PARAMS = pltpu.CompilerParams(use_tc_tiling_on_sc=False)


def _preload_packed(packed_hbm, pads_hbm, idx_p, t):
    pltpu.sync_copy(packed_hbm.at[0, pl.ds(t * BASE * CHUNK, BASE * CHUNK)],
                    idx_p.at[pl.ds(0, BASE * CHUNK)])

    @pl.when(t < EXTRA)
    def _():
        pltpu.sync_copy(
            packed_hbm.at[0, pl.ds((BASE * NTILES + t) * CHUNK, CHUNK)],
            idx_p.at[pl.ds(BASE * CHUNK, CHUNK)])

    @pl.when(t >= EXTRA)
    def _():
        pltpu.sync_copy(pads_hbm, idx_p.at[pl.ds(BASE * CHUNK, CHUNK)])

    for j in range(BASE + 1, CH):
        pltpu.sync_copy(pads_hbm, idx_p.at[pl.ds(j * CHUNK, CHUNK)])


def _degree_kernel():
    mesh = plsc.VectorSubcoreMesh(core_axis_name="c", subcore_axis_name="s")

    @functools.partial(
        pl.kernel,
        out_type=jax.ShapeDtypeStruct((2, N_PAD, 16), jnp.float32),
        mesh=mesh,
        compiler_params=_SC_PARAMS,
        scratch_types=[
            pltpu.VMEM((CH * CHUNK,), jnp.int32),
            pltpu.VMEM((4, CHUNK), jnp.int32),
            pltpu.VMEM((CHUNK, 16), jnp.float32),
            pltpu.VMEM_SHARED((N_PAD, 16), jnp.float32),
            pltpu.SemaphoreType.DMA,
            pltpu.SemaphoreType.DMA,
            pltpu.SemaphoreType.DMA,
            pltpu.SemaphoreType.DMA,
        ],
    )
    def deg(packed_hbm, ones_hbm, zeros_hbm, pads_hbm, out_hbm,
            idx_p, st_c, buf, acc, s0, s1, s2, s3):
        sems = [s0, s1, s2, s3]
        cid = lax.axis_index("c")
        sid = lax.axis_index("s")
        t = cid * 16 + sid

        pltpu.sync_copy(zeros_hbm, buf)
        for j in range(ROWS_PER_TILE // CHUNK):
            pltpu.sync_copy(buf, acc.at[pl.ds(sid * ROWS_PER_TILE + j * CHUNK, CHUNK)])
        _preload_packed(packed_hbm, pads_hbm, idx_p, t)
        pltpu.sync_copy(ones_hbm, buf)
        plsc.subcore_barrier()

        def unpack_c(i, k):
            for j in range(CHUNK // 16):
                v = idx_p[pl.ds(i * CHUNK + j * 16, 16)]
                st_c[k, pl.ds(j * 16, 16)] = jnp.right_shift(v, 16)

        def issue(k):
            pltpu.async_copy(buf, acc.at[st_c.at[k]], sems[k], add=True)

        def drain(k):
            pltpu.make_async_copy(buf, acc.at[st_c.at[k]], sems[k]).wait()

        for k in range(4):
            unpack_c(k, k)
            issue(k)

        def body(s, carry):
            for k in range(4):
                drain(k)
                unpack_c(4 * s + k, k)
                issue(k)
            return carry

        lax.fori_loop(1, CH // 4, body, 0)
        for k in range(4):
            drain(k)
        plsc.subcore_barrier()

        for j in range(ROWS_PER_TILE // CHUNK):
            off = sid * ROWS_PER_TILE + j * CHUNK
            pltpu.sync_copy(acc.at[pl.ds(off, CHUNK)], buf)
            pltpu.sync_copy(buf, out_hbm.at[cid, pl.ds(off, CHUNK)])

    return deg


def _mp_kernel(D, spm_table=False):
    mesh = plsc.VectorSubcoreMesh(core_axis_name="c", subcore_axis_name="s")
    tshapes = [pltpu.VMEM_SHARED((N, D), jnp.float32)] if spm_table else []

    @functools.partial(
        pl.kernel,
        out_type=jax.ShapeDtypeStruct((2, N_PAD, D), jnp.float32),
        mesh=mesh,
        compiler_params=_SC_PARAMS,
        scratch_types=[
            pltpu.VMEM((CH * CHUNK,), jnp.int32),
            pltpu.VMEM((2, CHUNK), jnp.int32),
            pltpu.VMEM((2, CHUNK), jnp.int32),
            pltpu.VMEM((CHUNK, D), jnp.float32),
            pltpu.VMEM((CHUNK, D), jnp.float32),
            pltpu.VMEM_SHARED((N_PAD, D), jnp.float32),
        ] + tshapes + [
            pltpu.SemaphoreType.DMA,
            pltpu.SemaphoreType.DMA,
            pltpu.SemaphoreType.DMA,
            pltpu.SemaphoreType.DMA,
        ],
    )
    def mp(packed_hbm, table_hbm, zeros_hbm, pads_hbm, out_hbm,
           idx_p, st_r, st_c, r0, r1, acc, *rest):
        if spm_table:
            table_spm = rest[0]
            g0, g1, t0, t1 = rest[1:]
        else:
            g0, g1, t0, t1 = rest
        rows = [r0, r1]
        semg = [g0, g1]
        sems = [t0, t1]
        cid = lax.axis_index("c")
        sid = lax.axis_index("s")
        t = cid * 16 + sid

        _preload_packed(packed_hbm, pads_hbm, idx_p, t)

        if spm_table:
            tb = N // 16
            for j in range(tb // CHUNK + 1):
                nrows = CHUNK if (j + 1) * CHUNK <= tb else tb - j * CHUNK
                off = sid * tb + j * CHUNK
                pltpu.sync_copy(table_hbm.at[pl.ds(off, nrows)],
                                r0.at[pl.ds(0, nrows)])
                pltpu.sync_copy(r0.at[pl.ds(0, nrows)],
                                table_spm.at[pl.ds(off, nrows)])
            gather_src = table_spm
        else:
            gather_src = table_hbm

        pltpu.sync_copy(zeros_hbm, r0)
        for j in range(ROWS_PER_TILE // CHUNK):
            pltpu.sync_copy(r0, acc.at[pl.ds(sid * ROWS_PER_TILE + j * CHUNK, CHUNK)])
        plsc.subcore_barrier()

        def unpack(i, b):
            for j in range(CHUNK // 16):
                v = idx_p[pl.ds(i * CHUNK + j * 16, 16)]
                st_r[b, pl.ds(j * 16, 16)] = jnp.bitwise_and(v, 0xFFFF)
                st_c[b, pl.ds(j * 16, 16)] = jnp.right_shift(v, 16)

        def issue_gather(b):
            pltpu.async_copy(gather_src.at[st_r.at[b]], rows[b], semg[b])

        def wait_gather(b):
            pltpu.make_async_copy(gather_src.at[st_r.at[b]], rows[b],
                                  semg[b]).wait()

        def issue_scatter(b):
            pltpu.async_copy(rows[b], acc.at[st_c.at[b]], sems[b], add=True)

        def wait_scatter(b):
            pltpu.make_async_copy(rows[b], acc.at[st_c.at[b]], sems[b]).wait()

        unpack(0, 0)
        issue_gather(0)
        unpack(1, 1)
        issue_gather(1)

        def body(s, carry):
            for k in range(2):
                i = 2 * s + k
                wait_gather(k)
                issue_scatter(k)
                wait_scatter(k)
                unpack(i + 2, k)
                issue_gather(k)
            return carry

        lax.fori_loop(0, (CH - 2) // 2, body, 0)

        for k in range(2):
            wait_gather(k)
            issue_scatter(k)
        for k in range(2):
            wait_scatter(k)
        plsc.subcore_barrier()

        for j in range(ROWS_PER_TILE // CHUNK):
            off = sid * ROWS_PER_TILE + j * CHUNK
            pltpu.sync_copy(acc.at[pl.ds(off, CHUNK)], r0)
            pltpu.sync_copy(r0, out_hbm.at[cid, pl.ds(off, CHUNK)])

    return mp


def _dis(degp0, degp1):
    deg = degp0[:, 0:1] + degp1[:, 0:1] + 1.0
    return lax.rsqrt(deg)


def _tc_pack(e_ref, out_ref):
    out_ref[...] = e_ref[0:1, :] + e_ref[1:2, :] * 65536


def _tc_a(degp_ref, x_ref, w_ref, out_ref):
    dis = _dis(degp_ref[0], degp_ref[1])
    out_ref[...] = dis * jnp.dot(x_ref[...], w_ref[...],
                                 preferred_element_type=jnp.float32)


def _tc_b(degp_ref, acc_ref, tab_ref, b_ref, w_ref, out_ref):
    dis = _dis(degp_ref[0], degp_ref[1])
    s = acc_ref[0] + acc_ref[1] + tab_ref[...]
    h = jnp.maximum(dis * s + b_ref[...], 0.0)
    out_ref[...] = dis * jnp.dot(h, w_ref[...],
                                 preferred_element_type=jnp.float32)


def _tc_c(degp_ref, acc_ref, tab_ref, b_ref, w_ref, bd_ref, out_ref):
    dis = _dis(degp_ref[0], degp_ref[1])
    s = acc_ref[0] + acc_ref[1] + tab_ref[...]
    emb = jnp.maximum(dis * s + b_ref[...], 0.0)
    out_ref[...] = jnp.dot(emb, w_ref[...],
                           preferred_element_type=jnp.float32) + bd_ref[...]


def _row_blocked(d):
    return pl.BlockSpec((RB, d), lambda i: (i, 0))


def _deg_spec():
    return pl.BlockSpec((2, RB, 16), lambda i: (0, i, 0))


def _acc_spec(d):
    return pl.BlockSpec((2, RB, d), lambda i: (0, i, 0))


def _full(shape):
    return pl.BlockSpec(shape, lambda i: tuple(0 for _ in shape))


@jax.jit
def kernel(x, edge_index, W1, b1, W2, b2, Wd, bd):
    f32 = jnp.float32
    i32 = jnp.int32
    pads_p = jnp.full((CHUNK,), TRASH * 65536, i32)

    ones16 = jnp.ones((CHUNK, 16), f32)
    zeros16 = jnp.zeros((CHUNK, 16), f32)
    zeros128 = jnp.zeros((CHUNK, 128), f32)
    zeros64 = jnp.zeros((CHUNK, 64), f32)

    eblk = E // 25
    packed = pl.pallas_call(
        _tc_pack,
        grid=(25,),
        in_specs=[pl.BlockSpec((2, eblk), lambda i: (0, i))],
        out_specs=pl.BlockSpec((1, eblk), lambda i: (0, i)),
        out_shape=jax.ShapeDtypeStruct((1, E), i32),
    )(edge_index)

    degp = _degree_kernel()(packed, ones16, zeros16, pads_p)

    grid = (N // RB,)
    table1 = pl.pallas_call(
        _tc_a,
        grid=grid,
        in_specs=[_deg_spec(), _row_blocked(D_IN), _full((D_IN, 128))],
        out_specs=_row_blocked(128),
        out_shape=jax.ShapeDtypeStruct((N, 128), f32),
    )(degp, x, W1.T)

    acc1 = _mp_kernel(128)(packed, table1, zeros128, pads_p)

    table2 = pl.pallas_call(
        _tc_b,
        grid=grid,
        in_specs=[_deg_spec(), _acc_spec(128), _row_blocked(128),
                  _full((1, 128)), _full((128, 64))],
        out_specs=_row_blocked(64),
        out_shape=jax.ShapeDtypeStruct((N, 64), f32),
    )(degp, acc1, table1, b1.reshape(1, 128), W2.T)

    acc2 = _mp_kernel(64, spm_table=True)(packed, table2, zeros64, pads_p)

    dout = 256
    wdt = jnp.zeros((64, dout), f32).at[:, :D_IN].set(Wd.T)
    bd_pad = jnp.zeros((1, dout), f32).at[0, :D_IN].set(bd)
    out = pl.pallas_call(
        _tc_c,
        grid=grid,
        in_specs=[_deg_spec(), _acc_spec(64), _row_blocked(64),
                  _full((1, 64)), _full((64, dout)), _full((1, dout))],
        out_specs=_row_blocked(dout),
        out_shape=jax.ShapeDtypeStruct((N, dout), f32),
    )(degp, acc2, table2, b2.reshape(1, 64), wdt, bd_pad)

    return out[:, :D_IN]


# speedup vs baseline: 1.6573x; 1.4046x over previous
import functools

import jax
import jax.numpy as jnp
from jax import lax
from jax.experimental import pallas as pl
from jax.experimental.pallas import tpu as pltpu
from jax.experimental.pallas import tpu_sc as plsc

N = 10000
E = 320000
D_IN = 165

N_PAD = 10240
TRASH = N
NTILES = 32
CHUNK = 128
RC = E // CHUNK
BASE = RC // NTILES
EXTRA = RC - BASE * NTILES
CH = 80
DW = 64
ROWS_PER_TILE = N_PAD // 16
RB = 400
_SC_PARAMS = pltpu.CompilerParams(use_tc_tiling_on_sc=False)


def _preload_packed(packed_hbm, pads_hbm, idx_p, t):
    pltpu.sync_copy(packed_hbm.at[0, pl.ds(t * BASE * CHUNK, BASE * CHUNK)],
                    idx_p.at[pl.ds(0, BASE * CHUNK)])

    @pl.when(t < EXTRA)
    def _():
        pltpu.sync_copy(
            packed_hbm.at[0, pl.ds((BASE * NTILES + t) * CHUNK, CHUNK)],
            idx_p.at[pl.ds(BASE * CHUNK, CHUNK)])

    @pl.when(t >= EXTRA)
    def _():
        pltpu.sync_copy(pads_hbm, idx_p.at[pl.ds(BASE * CHUNK, CHUNK)])

    for j in range(BASE + 1, CH):
        pltpu.sync_copy(pads_hbm, idx_p.at[pl.ds(j * CHUNK, CHUNK)])


def _degree_kernel():
    mesh = plsc.VectorSubcoreMesh(core_axis_name="c", subcore_axis_name="s")

    @functools.partial(
        pl.kernel,
        out_type=jax.ShapeDtypeStruct((2, N_PAD, 16), jnp.float32),
        mesh=mesh,
        compiler_params=_SC_PARAMS,
        scratch_types=[
            pltpu.VMEM((CH * CHUNK,), jnp.int32),
            pltpu.VMEM((4, CHUNK), jnp.int32),
            pltpu.VMEM((CHUNK, 16), jnp.float32),
            pltpu.VMEM_SHARED((N_PAD, 16), jnp.float32),
            pltpu.SemaphoreType.DMA,
            pltpu.SemaphoreType.DMA,
            pltpu.SemaphoreType.DMA,
            pltpu.SemaphoreType.DMA,
        ],
    )
    def deg(packed_hbm, ones_hbm, zeros_hbm, pads_hbm, out_hbm,
            idx_p, st_c, buf, acc, s0, s1, s2, s3):
        sems = [s0, s1, s2, s3]
        cid = lax.axis_index("c")
        sid = lax.axis_index("s")
        t = cid * 16 + sid

        pltpu.sync_copy(zeros_hbm, buf)
        for j in range(ROWS_PER_TILE // CHUNK):
            pltpu.sync_copy(buf, acc.at[pl.ds(sid * ROWS_PER_TILE + j * CHUNK, CHUNK)])
        _preload_packed(packed_hbm, pads_hbm, idx_p, t)
        pltpu.sync_copy(ones_hbm, buf)
        plsc.subcore_barrier()

        def unpack_c(i, k):
            for j in range(CHUNK // 16):
                v = idx_p[pl.ds(i * CHUNK + j * 16, 16)]
                st_c[k, pl.ds(j * 16, 16)] = jnp.right_shift(v, 16)

        def issue(k):
            pltpu.async_copy(buf, acc.at[st_c.at[k]], sems[k], add=True)

        def drain(k):
            pltpu.make_async_copy(buf, acc.at[st_c.at[k]], sems[k]).wait()

        for k in range(4):
            unpack_c(k, k)
            issue(k)

        def body(s, carry):
            for k in range(4):
                drain(k)
                unpack_c(4 * s + k, k)
                issue(k)
            return carry

        lax.fori_loop(1, CH // 4, body, 0)
        for k in range(4):
            drain(k)
        plsc.subcore_barrier()

        for j in range(ROWS_PER_TILE // CHUNK):
            off = sid * ROWS_PER_TILE + j * CHUNK
            pltpu.sync_copy(acc.at[pl.ds(off, CHUNK)], buf)
            pltpu.sync_copy(buf, out_hbm.at[cid, pl.ds(off, CHUNK)])

    return deg


def _mp_kernel(nt):
    mesh = plsc.VectorSubcoreMesh(core_axis_name="c", subcore_axis_name="s")

    @functools.partial(
        pl.kernel,
        out_type=jax.ShapeDtypeStruct((2, nt, N_PAD, DW), jnp.float32),
        mesh=mesh,
        compiler_params=_SC_PARAMS,
        scratch_types=[
            pltpu.VMEM((CH * CHUNK,), jnp.int32),
            pltpu.VMEM((2, CHUNK), jnp.int32),
            pltpu.VMEM((2, CHUNK), jnp.int32),
            pltpu.VMEM((CHUNK, DW), jnp.float32),
            pltpu.VMEM((CHUNK, DW), jnp.float32),
            pltpu.VMEM_SHARED((N_PAD, DW), jnp.float32),
            pltpu.VMEM_SHARED((N, DW), jnp.float32),
            pltpu.SemaphoreType.DMA,
            pltpu.SemaphoreType.DMA,
            pltpu.SemaphoreType.DMA,
            pltpu.SemaphoreType.DMA,
        ],
    )
    def mp(packed_hbm, *rest):
        tables = rest[:nt]
        zeros_hbm, pads_hbm, out_hbm = rest[nt:nt + 3]
        idx_p, st_r, st_c, r0, r1, acc, table_spm = rest[nt + 3:nt + 10]
        g0, g1, t0, t1 = rest[nt + 10:nt + 14]
        rows = [r0, r1]
        semg = [g0, g1]
        sems = [t0, t1]
        cid = lax.axis_index("c")
        sid = lax.axis_index("s")
        t = cid * 16 + sid

        _preload_packed(packed_hbm, pads_hbm, idx_p, t)

        def unpack(i, b):
            for j in range(CHUNK // 16):
                v = idx_p[pl.ds(i * CHUNK + j * 16, 16)]
                st_r[b, pl.ds(j * 16, 16)] = jnp.bitwise_and(v, 0xFFFF)
                st_c[b, pl.ds(j * 16, 16)] = jnp.right_shift(v, 16)

        def issue_gather(b):
            pltpu.async_copy(table_spm.at[st_r.at[b]], rows[b], semg[b])

        def wait_gather(b):
            pltpu.make_async_copy(table_spm.at[st_r.at[b]], rows[b],
                                  semg[b]).wait()

        def issue_scatter(b):
            pltpu.async_copy(rows[b], acc.at[st_c.at[b]], sems[b], add=True)

        def wait_scatter(b):
            pltpu.make_async_copy(rows[b], acc.at[st_c.at[b]], sems[b]).wait()

        for h in range(nt):
            tb = N // 16
            for j in range(tb // CHUNK + 1):
                nrows = CHUNK if (j + 1) * CHUNK <= tb else tb - j * CHUNK
                off = sid * tb + j * CHUNK
                pltpu.sync_copy(tables[h].at[pl.ds(off, nrows)],
                                r0.at[pl.ds(0, nrows)])
                pltpu.sync_copy(r0.at[pl.ds(0, nrows)],
                                table_spm.at[pl.ds(off, nrows)])

            pltpu.sync_copy(zeros_hbm, r0)
            for j in range(ROWS_PER_TILE // CHUNK):
                pltpu.sync_copy(
                    r0, acc.at[pl.ds(sid * ROWS_PER_TILE + j * CHUNK, CHUNK)])
            plsc.subcore_barrier()

            unpack(0, 0)
            issue_gather(0)
            unpack(1, 1)
            issue_gather(1)

            def body(s, carry):
                for k in range(2):
                    i = 2 * s + k
                    wait_gather(k)
                    issue_scatter(k)
                    wait_scatter(k)
                    unpack(i + 2, k)
                    issue_gather(k)
                return carry

            lax.fori_loop(0, (CH - 2) // 2, body, 0)

            for k in range(2):
                wait_gather(k)
                issue_scatter(k)
            for k in range(2):
                wait_scatter(k)
            plsc.subcore_barrier()

            for j in range(ROWS_PER_TILE // CHUNK):
                off = sid * ROWS_PER_TILE + j * CHUNK
                pltpu.sync_copy(acc.at[pl.ds(off, CHUNK)], r0)
                pltpu.sync_copy(r0, out_hbm.at[cid, h, pl.ds(off, CHUNK)])

    return mp


def _dis(degp0, degp1):
    deg = degp0[:, 0:1] + degp1[:, 0:1] + 1.0
    return lax.rsqrt(deg)


def _tc_pack(e_ref, out_ref):
    out_ref[...] = e_ref[0:1, :] + e_ref[1:2, :] * 65536


def _tc_a(degp_ref, x_ref, w_ref, outa_ref, outb_ref):
    dis = _dis(degp_ref[0], degp_ref[1])
    xw = jnp.dot(x_ref[...], w_ref[...], preferred_element_type=jnp.float32)
    outa_ref[...] = dis * xw[:, :DW]
    outb_ref[...] = dis * xw[:, DW:]


def _tc_b(degp_ref, acc_ref, taba_ref, tabb_ref, b_ref, w_ref, out_ref):
    dis = _dis(degp_ref[0], degp_ref[1])
    b = b_ref[...]
    w = w_ref[...]
    sa = acc_ref[0, 0] + acc_ref[1, 0] + taba_ref[...]
    sb = acc_ref[0, 1] + acc_ref[1, 1] + tabb_ref[...]
    ha = jnp.maximum(dis * sa + b[:, :DW], 0.0)
    hb = jnp.maximum(dis * sb + b[:, DW:], 0.0)
    out_ref[...] = dis * (
        jnp.dot(ha, w[:DW], preferred_element_type=jnp.float32)
        + jnp.dot(hb, w[DW:], preferred_element_type=jnp.float32))


def _tc_c(degp_ref, acc_ref, tab_ref, b_ref, w_ref, bd_ref, out_ref):
    dis = _dis(degp_ref[0], degp_ref[1])
    s = acc_ref[0, 0] + acc_ref[1, 0] + tab_ref[...]
    emb = jnp.maximum(dis * s + b_ref[...], 0.0)
    out_ref[...] = jnp.dot(emb, w_ref[...],
                           preferred_element_type=jnp.float32) + bd_ref[...]


def _row_blocked(d):
    return pl.BlockSpec((RB, d), lambda i: (i, 0))


def _deg_spec():
    return pl.BlockSpec((2, RB, 16), lambda i: (0, i, 0))


def _acc_spec(nt):
    return pl.BlockSpec((2, nt, RB, DW), lambda i: (0, 0, i, 0))


def _full(shape):
    return pl.BlockSpec(shape, lambda i: tuple(0 for _ in shape))


@jax.jit
def kernel(x, edge_index, W1, b1, W2, b2, Wd, bd):
    f32 = jnp.float32
    i32 = jnp.int32
    pads_p = jnp.full((CHUNK,), TRASH * 65536, i32)

    ones16 = jnp.ones((CHUNK, 16), f32)
    zeros16 = jnp.zeros((CHUNK, 16), f32)
    zeros128 = jnp.zeros((CHUNK, 128), f32)
    zeros64 = jnp.zeros((CHUNK, 64), f32)

    eblk = E // 25
    packed = pl.pallas_call(
        _tc_pack,
        grid=(25,),
        in_specs=[pl.BlockSpec((2, eblk), lambda i: (0, i))],
        out_specs=pl.BlockSpec((1, eblk), lambda i: (0, i)),
        out_shape=jax.ShapeDtypeStruct((1, E), i32),
    )(edge_index)

    degp = _degree_kernel()(packed, ones16, zeros16, pads_p)

    grid = (N // RB,)
    table1a, table1b = pl.pallas_call(
        _tc_a,
        grid=grid,
        in_specs=[_deg_spec(), _row_blocked(D_IN), _full((D_IN, 128))],
        out_specs=[_row_blocked(DW), _row_blocked(DW)],
        out_shape=[jax.ShapeDtypeStruct((N, DW), f32)] * 2,
    )(degp, x, W1.T)

    acc1 = _mp_kernel(2)(packed, table1a, table1b, zeros64, pads_p)

    table2 = pl.pallas_call(
        _tc_b,
        grid=grid,
        in_specs=[_deg_spec(), _acc_spec(2), _row_blocked(DW),
                  _row_blocked(DW), _full((1, 128)), _full((128, DW))],
        out_specs=_row_blocked(DW),
        out_shape=jax.ShapeDtypeStruct((N, DW), f32),
    )(degp, acc1, table1a, table1b, b1.reshape(1, 128), W2.T)

    acc2 = _mp_kernel(1)(packed, table2, zeros64, pads_p)

    dout = 256
    wdt = jnp.zeros((DW, dout), f32).at[:, :D_IN].set(Wd.T)
    bd_pad = jnp.zeros((1, dout), f32).at[0, :D_IN].set(bd)
    out = pl.pallas_call(
        _tc_c,
        grid=grid,
        in_specs=[_deg_spec(), _acc_spec(1), _row_blocked(DW),
                  _full((1, DW)), _full((DW, dout)), _full((1, dout))],
        out_specs=_row_blocked(dout),
        out_shape=jax.ShapeDtypeStruct((N, dout), f32),
    )(degp, acc2, table2, b2.reshape(1, DW), wdt, bd_pad)

    return out[:, :D_IN]


# speedup vs baseline: 1.6746x; 1.0105x over previous
import functools

import jax
import jax.numpy as jnp
from jax import lax
from jax.experimental import pallas as pl
from jax.experimental.pallas import tpu as pltpu
from jax.experimental.pallas import tpu_sc as plsc

N = 10000
E = 320000
D_IN = 165

N_PAD = 10240
TRASH = N
NTILES = 32
CHUNK = 128
RC = E // CHUNK
BASE = RC // NTILES
EXTRA = RC - BASE * NTILES
CH = 80
DW = 64
ROWS_PER_TILE = N_PAD // 16
RB = 400
_SC_PARAMS = pltpu.CompilerParams(use_tc_tiling_on_sc=False)


def _preload_flat(slice_fn, pads_hbm, idx_p, t):
    pltpu.sync_copy(slice_fn(t * BASE * CHUNK, BASE * CHUNK),
                    idx_p.at[pl.ds(0, BASE * CHUNK)])

    @pl.when(t < EXTRA)
    def _():
        pltpu.sync_copy(slice_fn((BASE * NTILES + t) * CHUNK, CHUNK),
                        idx_p.at[pl.ds(BASE * CHUNK, CHUNK)])

    @pl.when(t >= EXTRA)
    def _():
        pltpu.sync_copy(pads_hbm, idx_p.at[pl.ds(BASE * CHUNK, CHUNK)])

    for j in range(BASE + 1, CH):
        pltpu.sync_copy(pads_hbm, idx_p.at[pl.ds(j * CHUNK, CHUNK)])


def _degree_kernel():
    mesh = plsc.VectorSubcoreMesh(core_axis_name="c", subcore_axis_name="s")

    @functools.partial(
        pl.kernel,
        out_type=jax.ShapeDtypeStruct((2, N_PAD, 16), jnp.float32),
        mesh=mesh,
        compiler_params=_SC_PARAMS,
        scratch_types=[
            pltpu.VMEM((CH * CHUNK,), jnp.int32),
            pltpu.VMEM((4, CHUNK), jnp.int32),
            pltpu.VMEM((CHUNK, 16), jnp.float32),
            pltpu.VMEM_SHARED((N_PAD, 16), jnp.float32),
            pltpu.SemaphoreType.DMA,
            pltpu.SemaphoreType.DMA,
            pltpu.SemaphoreType.DMA,
            pltpu.SemaphoreType.DMA,
        ],
    )
    def deg(edges_hbm, ones_hbm, zeros_hbm, pads_hbm, out_hbm,
            idx_p, st_c, buf, acc, s0, s1, s2, s3):
        sems = [s0, s1, s2, s3]
        cid = lax.axis_index("c")
        sid = lax.axis_index("s")
        t = cid * 16 + sid

        pltpu.sync_copy(zeros_hbm, buf)
        for j in range(ROWS_PER_TILE // CHUNK):
            pltpu.sync_copy(buf, acc.at[pl.ds(sid * ROWS_PER_TILE + j * CHUNK, CHUNK)])
        _preload_flat(lambda off, n: edges_hbm.at[1, pl.ds(off, n)],
                      pads_hbm, idx_p, t)
        pltpu.sync_copy(ones_hbm, buf)
        plsc.subcore_barrier()

        def unpack_c(i, k):
            for j in range(CHUNK // 16):
                st_c[k, pl.ds(j * 16, 16)] = idx_p[pl.ds(i * CHUNK + j * 16, 16)]

        def issue(k):
            pltpu.async_copy(buf, acc.at[st_c.at[k]], sems[k], add=True)

        def drain(k):
            pltpu.make_async_copy(buf, acc.at[st_c.at[k]], sems[k]).wait()

        for k in range(4):
            unpack_c(k, k)
            issue(k)

        def body(s, carry):
            for k in range(4):
                drain(k)
                unpack_c(4 * s + k, k)
                issue(k)
            return carry

        lax.fori_loop(1, CH // 4, body, 0)
        for k in range(4):
            drain(k)
        plsc.subcore_barrier()

        for j in range(ROWS_PER_TILE // CHUNK):
            off = sid * ROWS_PER_TILE + j * CHUNK
            pltpu.sync_copy(acc.at[pl.ds(off, CHUNK)], buf)
            pltpu.sync_copy(buf, out_hbm.at[cid, pl.ds(off, CHUNK)])

    return deg


def _mp_kernel(nt):
    mesh = plsc.VectorSubcoreMesh(core_axis_name="c", subcore_axis_name="s")

    @functools.partial(
        pl.kernel,
        out_type=jax.ShapeDtypeStruct((2, nt, N_PAD, DW), jnp.float32),
        mesh=mesh,
        compiler_params=_SC_PARAMS,
        scratch_types=[
            pltpu.VMEM((CH * CHUNK,), jnp.int32),
            pltpu.VMEM((2, CHUNK), jnp.int32),
            pltpu.VMEM((2, CHUNK), jnp.int32),
            pltpu.VMEM((CHUNK, DW), jnp.float32),
            pltpu.VMEM((CHUNK, DW), jnp.float32),
            pltpu.VMEM_SHARED((N_PAD, DW), jnp.float32),
            pltpu.VMEM_SHARED((N, DW), jnp.float32),
            pltpu.SemaphoreType.DMA,
            pltpu.SemaphoreType.DMA,
            pltpu.SemaphoreType.DMA,
            pltpu.SemaphoreType.DMA,
        ],
    )
    def mp(packed_hbm, *rest):
        tables = rest[:nt]
        zeros_hbm, pads_hbm, out_hbm = rest[nt:nt + 3]
        idx_p, st_r, st_c, r0, r1, acc, table_spm = rest[nt + 3:nt + 10]
        g0, g1, t0, t1 = rest[nt + 10:nt + 14]
        rows = [r0, r1]
        semg = [g0, g1]
        sems = [t0, t1]
        cid = lax.axis_index("c")
        sid = lax.axis_index("s")
        t = cid * 16 + sid

        _preload_flat(lambda off, n: packed_hbm.at[0, pl.ds(off, n)],
                      pads_hbm, idx_p, t)

        def unpack(i, b):
            for j in range(CHUNK // 16):
                v = idx_p[pl.ds(i * CHUNK + j * 16, 16)]
                st_r[b, pl.ds(j * 16, 16)] = jnp.bitwise_and(v, 0xFFFF)
                st_c[b, pl.ds(j * 16, 16)] = jnp.right_shift(v, 16)

        def issue_gather(b):
            pltpu.async_copy(table_spm.at[st_r.at[b]], rows[b], semg[b])

        def wait_gather(b):
            pltpu.make_async_copy(table_spm.at[st_r.at[b]], rows[b],
                                  semg[b]).wait()

        def issue_scatter(b):
            pltpu.async_copy(rows[b], acc.at[st_c.at[b]], sems[b], add=True)

        def wait_scatter(b):
            pltpu.make_async_copy(rows[b], acc.at[st_c.at[b]], sems[b]).wait()

        for h in range(nt):
            tb = N // 16
            for j in range(tb // CHUNK + 1):
                nrows = CHUNK if (j + 1) * CHUNK <= tb else tb - j * CHUNK
                off = sid * tb + j * CHUNK
                pltpu.sync_copy(tables[h].at[pl.ds(off, nrows)],
                                r0.at[pl.ds(0, nrows)])
                pltpu.sync_copy(r0.at[pl.ds(0, nrows)],
                                table_spm.at[pl.ds(off, nrows)])

            pltpu.sync_copy(zeros_hbm, r0)
            for j in range(ROWS_PER_TILE // CHUNK):
                pltpu.sync_copy(
                    r0, acc.at[pl.ds(sid * ROWS_PER_TILE + j * CHUNK, CHUNK)])
            plsc.subcore_barrier()

            unpack(0, 0)
            issue_gather(0)
            unpack(1, 1)
            issue_gather(1)

            def body(s, carry):
                for k in range(2):
                    i = 2 * s + k
                    wait_gather(k)
                    issue_scatter(k)
                    wait_scatter(k)
                    unpack(i + 2, k)
                    issue_gather(k)
                return carry

            lax.fori_loop(0, (CH - 2) // 2, body, 0)

            for k in range(2):
                wait_gather(k)
                issue_scatter(k)
            for k in range(2):
                wait_scatter(k)
            plsc.subcore_barrier()

            for j in range(ROWS_PER_TILE // CHUNK):
                off = sid * ROWS_PER_TILE + j * CHUNK
                pltpu.sync_copy(acc.at[pl.ds(off, CHUNK)], r0)
                pltpu.sync_copy(r0, out_hbm.at[cid, h, pl.ds(off, CHUNK)])

    return mp


def _dis(degp0, degp1):
    deg = degp0[:, 0:1] + degp1[:, 0:1] + 1.0
    return lax.rsqrt(deg)


def _tc_a(degp_ref, x_ref, w_ref, e_ref, outa_ref, outb_ref, packed_ref):
    dis = _dis(degp_ref[0], degp_ref[1])
    xw = jnp.dot(x_ref[...], w_ref[...], preferred_element_type=jnp.float32)
    outa_ref[...] = dis * xw[:, :DW]
    outb_ref[...] = dis * xw[:, DW:]
    packed_ref[...] = e_ref[0:1, :] + e_ref[1:2, :] * 65536


def _tc_b(degp_ref, acc_ref, taba_ref, tabb_ref, b_ref, w_ref, out_ref):
    dis = _dis(degp_ref[0], degp_ref[1])
    b = b_ref[...]
    w = w_ref[...]
    sa = acc_ref[0, 0] + acc_ref[1, 0] + taba_ref[...]
    sb = acc_ref[0, 1] + acc_ref[1, 1] + tabb_ref[...]
    ha = jnp.maximum(dis * sa + b[:, :DW], 0.0)
    hb = jnp.maximum(dis * sb + b[:, DW:], 0.0)
    out_ref[...] = dis * (
        jnp.dot(ha, w[:DW], preferred_element_type=jnp.float32)
        + jnp.dot(hb, w[DW:], preferred_element_type=jnp.float32))


def _tc_c(degp_ref, acc_ref, tab_ref, b_ref, w_ref, bd_ref, out_ref):
    dis = _dis(degp_ref[0], degp_ref[1])
    s = acc_ref[0, 0] + acc_ref[1, 0] + tab_ref[...]
    emb = jnp.maximum(dis * s + b_ref[...], 0.0)
    out_ref[...] = jnp.dot(emb, w_ref[...],
                           preferred_element_type=jnp.float32) + bd_ref[...]


def _row_blocked(d):
    return pl.BlockSpec((RB, d), lambda i: (i, 0))


def _deg_spec():
    return pl.BlockSpec((2, RB, 16), lambda i: (0, i, 0))


def _acc_spec(nt):
    return pl.BlockSpec((2, nt, RB, DW), lambda i: (0, 0, i, 0))


def _full(shape):
    return pl.BlockSpec(shape, lambda i: tuple(0 for _ in shape))


@jax.jit
def kernel(x, edge_index, W1, b1, W2, b2, Wd, bd):
    f32 = jnp.float32
    i32 = jnp.int32
    pads_p = jnp.full((CHUNK,), TRASH * 65536, i32)
    pads_c = jnp.full((CHUNK,), TRASH, i32)

    ones16 = jnp.ones((CHUNK, 16), f32)
    zeros16 = jnp.zeros((CHUNK, 16), f32)
    zeros64 = jnp.zeros((CHUNK, 64), f32)

    degp = _degree_kernel()(edge_index, ones16, zeros16, pads_c)

    grid = (N // RB,)
    eblk = E // (N // RB)
    table1a, table1b, packed = pl.pallas_call(
        _tc_a,
        grid=grid,
        in_specs=[_deg_spec(), _row_blocked(D_IN), _full((D_IN, 128)),
                  pl.BlockSpec((2, eblk), lambda i: (0, i))],
        out_specs=[_row_blocked(DW), _row_blocked(DW),
                   pl.BlockSpec((1, eblk), lambda i: (0, i))],
        out_shape=[jax.ShapeDtypeStruct((N, DW), f32),
                   jax.ShapeDtypeStruct((N, DW), f32),
                   jax.ShapeDtypeStruct((1, E), i32)],
    )(degp, x, W1.T, edge_index)

    acc1 = _mp_kernel(2)(packed, table1a, table1b, zeros64, pads_p)

    table2 = pl.pallas_call(
        _tc_b,
        grid=grid,
        in_specs=[_deg_spec(), _acc_spec(2), _row_blocked(DW),
                  _row_blocked(DW), _full((1, 128)), _full((128, DW))],
        out_specs=_row_blocked(DW),
        out_shape=jax.ShapeDtypeStruct((N, DW), f32),
    )(degp, acc1, table1a, table1b, b1.reshape(1, 128), W2.T)

    acc2 = _mp_kernel(1)(packed, table2, zeros64, pads_p)

    dout = 256
    wdt = jnp.zeros((DW, dout), f32).at[:, :D_IN].set(Wd.T)
    bd_pad = jnp.zeros((1, dout), f32).at[0, :D_IN].set(bd)
    out = pl.pallas_call(
        _tc_c,
        grid=grid,
        in_specs=[_deg_spec(), _acc_spec(1), _row_blocked(DW),
                  _full((1, DW)), _full((DW, dout)), _full((1, dout))],
        out_specs=_row_blocked(dout),
        out_shape=jax.ShapeDtypeStruct((N, dout), f32),
    )(degp, acc2, table2, b2.reshape(1, DW), wdt, bd_pad)

    return out[:, :D_IN]


# speedup vs baseline: 1.8397x; 1.0985x over previous
import functools

import jax
import jax.numpy as jnp
from jax import lax
from jax.experimental import pallas as pl
from jax.experimental.pallas import tpu as pltpu
from jax.experimental.pallas import tpu_sc as plsc

N = 10000
E = 320000
D_IN = 165

N_PAD = 10240
TRASH = N
NTILES = 32
CHUNK = 128
RC = E // CHUNK
BASE = RC // NTILES
EXTRA = RC - BASE * NTILES
CH = 80
DW = 64
ROWS_PER_TILE = N_PAD // 16
RB = 400
_SC_PARAMS = pltpu.CompilerParams(use_tc_tiling_on_sc=False)


def _preload_flat(slice_fn, pads_hbm, idx_p, t):
    pltpu.sync_copy(slice_fn(t * BASE * CHUNK, BASE * CHUNK),
                    idx_p.at[pl.ds(0, BASE * CHUNK)])

    @pl.when(t < EXTRA)
    def _():
        pltpu.sync_copy(slice_fn((BASE * NTILES + t) * CHUNK, CHUNK),
                        idx_p.at[pl.ds(BASE * CHUNK, CHUNK)])

    @pl.when(t >= EXTRA)
    def _():
        pltpu.sync_copy(pads_hbm, idx_p.at[pl.ds(BASE * CHUNK, CHUNK)])

    for j in range(BASE + 1, CH):
        pltpu.sync_copy(pads_hbm, idx_p.at[pl.ds(j * CHUNK, CHUNK)])


def _degree_kernel():
    mesh = plsc.VectorSubcoreMesh(core_axis_name="c", subcore_axis_name="s")

    @functools.partial(
        pl.kernel,
        out_type=jax.ShapeDtypeStruct((2, N_PAD, 16), jnp.float32),
        mesh=mesh,
        compiler_params=_SC_PARAMS,
        scratch_types=[
            pltpu.VMEM((CH * CHUNK,), jnp.int32),
            pltpu.VMEM((4, CHUNK), jnp.int32),
            pltpu.VMEM((CHUNK, 16), jnp.float32),
            pltpu.VMEM_SHARED((N_PAD, 16), jnp.float32),
            pltpu.SemaphoreType.DMA,
            pltpu.SemaphoreType.DMA,
            pltpu.SemaphoreType.DMA,
            pltpu.SemaphoreType.DMA,
        ],
    )
    def deg(edges_hbm, ones_hbm, zeros_hbm, pads_hbm, out_hbm,
            idx_p, st_c, buf, acc, s0, s1, s2, s3):
        sems = [s0, s1, s2, s3]
        cid = lax.axis_index("c")
        sid = lax.axis_index("s")
        t = cid * 16 + sid

        pltpu.sync_copy(zeros_hbm, buf)
        for j in range(ROWS_PER_TILE // CHUNK):
            pltpu.sync_copy(buf, acc.at[pl.ds(sid * ROWS_PER_TILE + j * CHUNK, CHUNK)])
        _preload_flat(lambda off, n: edges_hbm.at[1, pl.ds(off, n)],
                      pads_hbm, idx_p, t)
        pltpu.sync_copy(ones_hbm, buf)
        plsc.subcore_barrier()

        def unpack_c(i, k):
            for j in range(CHUNK // 16):
                st_c[k, pl.ds(j * 16, 16)] = idx_p[pl.ds(i * CHUNK + j * 16, 16)]

        def issue(k):
            pltpu.async_copy(buf, acc.at[st_c.at[k]], sems[k], add=True)

        def drain(k):
            pltpu.make_async_copy(buf, acc.at[st_c.at[k]], sems[k]).wait()

        for k in range(4):
            unpack_c(k, k)
            issue(k)

        def body(s, carry):
            for k in range(4):
                drain(k)
                unpack_c(4 * s + k, k)
                issue(k)
            return carry

        lax.fori_loop(1, CH // 4, body, 0)
        for k in range(4):
            drain(k)
        plsc.subcore_barrier()

        for j in range(ROWS_PER_TILE // CHUNK):
            off = sid * ROWS_PER_TILE + j * CHUNK
            pltpu.sync_copy(acc.at[pl.ds(off, CHUNK)], buf)
            pltpu.sync_copy(buf, out_hbm.at[cid, pl.ds(off, CHUNK)])

    return deg


def _mp_kernel(nt):
    mesh = plsc.VectorSubcoreMesh(core_axis_name="c", subcore_axis_name="s")

    @functools.partial(
        pl.kernel,
        out_type=jax.ShapeDtypeStruct((2, nt, N_PAD, DW), jnp.float32),
        mesh=mesh,
        compiler_params=_SC_PARAMS,
        scratch_types=[
            pltpu.VMEM((CH * CHUNK,), jnp.int32),
            pltpu.VMEM((4, CHUNK), jnp.int32),
            pltpu.VMEM((4, CHUNK), jnp.int32),
            pltpu.VMEM((CHUNK, DW), jnp.float32),
            pltpu.VMEM((CHUNK, DW), jnp.float32),
            pltpu.VMEM((CHUNK, DW), jnp.float32),
            pltpu.VMEM((CHUNK, DW), jnp.float32),
            pltpu.VMEM_SHARED((N_PAD, DW), jnp.float32),
            pltpu.VMEM_SHARED((N, DW), jnp.float32),
        ] + [pltpu.SemaphoreType.DMA for _ in range(8)],
    )
    def mp(packed_hbm, *rest):
        tables = rest[:nt]
        zeros_hbm, pads_hbm, out_hbm = rest[nt:nt + 3]
        idx_p, st_r, st_c, r0, r1, r2, r3, acc, table_spm = rest[nt + 3:nt + 12]
        rows = [r0, r1, r2, r3]
        semg = rest[nt + 12:nt + 16]
        sems = rest[nt + 16:nt + 20]
        cid = lax.axis_index("c")
        sid = lax.axis_index("s")
        t = cid * 16 + sid

        _preload_flat(lambda off, n: packed_hbm.at[0, pl.ds(off, n)],
                      pads_hbm, idx_p, t)

        def unpack(i, b):
            for j in range(CHUNK // 16):
                v = idx_p[pl.ds(i * CHUNK + j * 16, 16)]
                st_r[b, pl.ds(j * 16, 16)] = jnp.bitwise_and(v, 0xFFFF)
                st_c[b, pl.ds(j * 16, 16)] = jnp.right_shift(v, 16)

        def issue_gather(b):
            pltpu.async_copy(table_spm.at[st_r.at[b]], rows[b], semg[b])

        def wait_gather(b):
            pltpu.make_async_copy(table_spm.at[st_r.at[b]], rows[b],
                                  semg[b]).wait()

        def issue_scatter(b):
            pltpu.async_copy(rows[b], acc.at[st_c.at[b]], sems[b], add=True)

        def wait_scatter(b):
            pltpu.make_async_copy(rows[b], acc.at[st_c.at[b]], sems[b]).wait()

        for h in range(nt):
            tb = N // 16
            for j in range(tb // CHUNK + 1):
                nrows = CHUNK if (j + 1) * CHUNK <= tb else tb - j * CHUNK
                off = sid * tb + j * CHUNK
                pltpu.sync_copy(tables[h].at[pl.ds(off, nrows)],
                                r0.at[pl.ds(0, nrows)])
                pltpu.sync_copy(r0.at[pl.ds(0, nrows)],
                                table_spm.at[pl.ds(off, nrows)])

            pltpu.sync_copy(zeros_hbm, r0)
            for j in range(ROWS_PER_TILE // CHUNK):
                pltpu.sync_copy(
                    r0, acc.at[pl.ds(sid * ROWS_PER_TILE + j * CHUNK, CHUNK)])
            plsc.subcore_barrier()

            unpack(0, 0)
            issue_gather(0)
            unpack(1, 1)
            issue_gather(1)
            for i in range(2):
                wait_gather(i)
                issue_scatter(i)
                unpack(i + 2, i + 2)
                issue_gather(i + 2)

            def body(s, carry):
                for k in range(4):
                    b = (2 + k) % 4
                    i = 4 * s + 2 + k
                    wait_gather(b)
                    issue_scatter(b)
                    wait_scatter((b + 2) % 4)
                    unpack(i + 2, (b + 2) % 4)
                    issue_gather((b + 2) % 4)
                return carry

            lax.fori_loop(0, (CH - 4) // 4, body, 0)

            for b in (2, 3):
                wait_gather(b)
                issue_scatter(b)
            for b in range(4):
                wait_scatter(b)
            plsc.subcore_barrier()

            for j in range(ROWS_PER_TILE // CHUNK):
                off = sid * ROWS_PER_TILE + j * CHUNK
                pltpu.sync_copy(acc.at[pl.ds(off, CHUNK)], r0)
                pltpu.sync_copy(r0, out_hbm.at[cid, h, pl.ds(off, CHUNK)])

    return mp


def _dis(degp0, degp1):
    deg = degp0[:, 0:1] + degp1[:, 0:1] + 1.0
    return lax.rsqrt(deg)


def _tc_a(degp_ref, x_ref, w_ref, e_ref, outa_ref, outb_ref, packed_ref):
    dis = _dis(degp_ref[0], degp_ref[1])
    xw = jnp.dot(x_ref[...], w_ref[...], preferred_element_type=jnp.float32)
    outa_ref[...] = dis * xw[:, :DW]
    outb_ref[...] = dis * xw[:, DW:]
    packed_ref[...] = e_ref[0:1, :] + e_ref[1:2, :] * 65536


def _tc_b(degp_ref, acc_ref, taba_ref, tabb_ref, b_ref, w_ref, out_ref):
    dis = _dis(degp_ref[0], degp_ref[1])
    b = b_ref[...]
    w = w_ref[...]
    sa = acc_ref[0, 0] + acc_ref[1, 0] + taba_ref[...]
    sb = acc_ref[0, 1] + acc_ref[1, 1] + tabb_ref[...]
    ha = jnp.maximum(dis * sa + b[:, :DW], 0.0)
    hb = jnp.maximum(dis * sb + b[:, DW:], 0.0)
    out_ref[...] = dis * (
        jnp.dot(ha, w[:DW], preferred_element_type=jnp.float32)
        + jnp.dot(hb, w[DW:], preferred_element_type=jnp.float32))


def _tc_c(degp_ref, acc_ref, tab_ref, b_ref, w_ref, bd_ref, out_ref):
    dis = _dis(degp_ref[0], degp_ref[1])
    s = acc_ref[0, 0] + acc_ref[1, 0] + tab_ref[...]
    emb = jnp.maximum(dis * s + b_ref[...], 0.0)
    out_ref[...] = jnp.dot(emb, w_ref[...],
                           preferred_element_type=jnp.float32) + bd_ref[...]


def _row_blocked(d):
    return pl.BlockSpec((RB, d), lambda i: (i, 0))


def _deg_spec():
    return pl.BlockSpec((2, RB, 16), lambda i: (0, i, 0))


def _acc_spec(nt):
    return pl.BlockSpec((2, nt, RB, DW), lambda i: (0, 0, i, 0))


def _full(shape):
    return pl.BlockSpec(shape, lambda i: tuple(0 for _ in shape))


@jax.jit
def kernel(x, edge_index, W1, b1, W2, b2, Wd, bd):
    f32 = jnp.float32
    i32 = jnp.int32
    pads_p = jnp.full((CHUNK,), TRASH * 65536, i32)
    pads_c = jnp.full((CHUNK,), TRASH, i32)

    ones16 = jnp.ones((CHUNK, 16), f32)
    zeros16 = jnp.zeros((CHUNK, 16), f32)
    zeros64 = jnp.zeros((CHUNK, 64), f32)

    degp = _degree_kernel()(edge_index, ones16, zeros16, pads_c)

    grid = (N // RB,)
    eblk = E // (N // RB)
    table1a, table1b, packed = pl.pallas_call(
        _tc_a,
        grid=grid,
        in_specs=[_deg_spec(), _row_blocked(D_IN), _full((D_IN, 128)),
                  pl.BlockSpec((2, eblk), lambda i: (0, i))],
        out_specs=[_row_blocked(DW), _row_blocked(DW),
                   pl.BlockSpec((1, eblk), lambda i: (0, i))],
        out_shape=[jax.ShapeDtypeStruct((N, DW), f32),
                   jax.ShapeDtypeStruct((N, DW), f32),
                   jax.ShapeDtypeStruct((1, E), i32)],
    )(degp, x, W1.T, edge_index)

    acc1 = _mp_kernel(2)(packed, table1a, table1b, zeros64, pads_p)

    table2 = pl.pallas_call(
        _tc_b,
        grid=grid,
        in_specs=[_deg_spec(), _acc_spec(2), _row_blocked(DW),
                  _row_blocked(DW), _full((1, 128)), _full((128, DW))],
        out_specs=_row_blocked(DW),
        out_shape=jax.ShapeDtypeStruct((N, DW), f32),
    )(degp, acc1, table1a, table1b, b1.reshape(1, 128), W2.T)

    acc2 = _mp_kernel(1)(packed, table2, zeros64, pads_p)

    dout = 256
    wdt = jnp.zeros((DW, dout), f32).at[:, :D_IN].set(Wd.T)
    bd_pad = jnp.zeros((1, dout), f32).at[0, :D_IN].set(bd)
    out = pl.pallas_call(
        _tc_c,
        grid=grid,
        in_specs=[_deg_spec(), _acc_spec(1), _row_blocked(DW),
                  _full((1, DW)), _full((DW, dout)), _full((1, dout))],
        out_specs=_row_blocked(dout),
        out_shape=jax.ShapeDtypeStruct((N, dout), f32),
    )(degp, acc2, table2, b2.reshape(1, DW), wdt, bd_pad)

    return out[:, :D_IN]
